# Initial kernel scaffold; baseline (speedup 1.0000x reference)
import jax
import jax.numpy as jnp
from jax.experimental import pallas as pl


def kernel(x, A, B, W, bias):
    raise NotImplementedError("write your pallas kernel here")


# speedup vs baseline: 4.7873x; 4.7873x over previous
import functools

import jax
import jax.numpy as jnp
import numpy as np
from jax.experimental import pallas as pl

NUM_ADAPTERS = 16
RANK = 16
D_IN = 1024
D_OUT = 1024
TOPK = 4
TOKENS = 2048
TOKEN_TILE = 256


def _routewrap_kernel(x_ref, atf_ref, wt_ref, bft_ref, bias_ref, s_ref, st_ref,
                      t_ref, t2_ref, out_ref):
    xt = x_ref[:]
    V = jnp.dot(xt, atf_ref[:], preferred_element_type=jnp.float32)
    n2 = jnp.dot(V * V, s_ref[:], preferred_element_type=jnp.float32)

    iota = jax.lax.broadcasted_iota(jnp.int32, n2.shape, 1)
    cur = n2
    mask = jnp.zeros_like(n2)
    for _ in range(TOPK):
        m = jnp.max(cur, axis=-1, keepdims=True)
        eq = cur == m
        ci = jnp.min(jnp.where(eq, iota, NUM_ADAPTERS), axis=-1, keepdims=True)
        oh = iota == ci
        mask = jnp.where(oh, mask + 1.0, mask)
        cur = jnp.where(oh, -1.0, cur)
    routes = mask * (1.0 / TOPK)

    M = jnp.dot(routes, st_ref[:], preferred_element_type=jnp.float32)
    vs = jnp.dot(M * V, t_ref[:], preferred_element_type=jnp.float32)
    U = M * jnp.dot(vs, t2_ref[:], preferred_element_type=jnp.float32)

    acc = jnp.dot(xt, wt_ref[:], preferred_element_type=jnp.float32)
    acc = acc + jnp.dot(U, bft_ref[:], preferred_element_type=jnp.float32)
    out_ref[:] = acc + bias_ref[:]


@jax.jit
def kernel(x, A, B, W, bias):
    b, s, _ = x.shape
    x2d = x.reshape(b * s, D_IN)
    atf = A.reshape(NUM_ADAPTERS * RANK, D_IN).T
    wt = W.T
    bft = B.transpose(0, 2, 1).reshape(NUM_ADAPTERS * RANK, D_OUT)
    bias2d = bias.reshape(1, D_OUT)

    blk = np.zeros((NUM_ADAPTERS * RANK, NUM_ADAPTERS), dtype=np.float32)
    blk[np.arange(NUM_ADAPTERS * RANK), np.arange(NUM_ADAPTERS * RANK) // RANK] = 1.0
    rnk = np.zeros((NUM_ADAPTERS * RANK, RANK), dtype=np.float32)
    rnk[np.arange(NUM_ADAPTERS * RANK), np.arange(NUM_ADAPTERS * RANK) % RANK] = 1.0
    S = jnp.asarray(blk)
    St = jnp.asarray(blk.T)
    T = jnp.asarray(rnk)
    T2 = jnp.asarray(rnk.T)

    n_tiles = (b * s) // TOKEN_TILE
    const = lambda i: (0, 0)
    out = pl.pallas_call(
        _routewrap_kernel,
        grid=(n_tiles,),
        in_specs=[
            pl.BlockSpec((TOKEN_TILE, D_IN), lambda i: (i, 0)),
            pl.BlockSpec((D_IN, NUM_ADAPTERS * RANK), const),
            pl.BlockSpec((D_IN, D_OUT), const),
            pl.BlockSpec((NUM_ADAPTERS * RANK, D_OUT), const),
            pl.BlockSpec((1, D_OUT), const),
            pl.BlockSpec((NUM_ADAPTERS * RANK, NUM_ADAPTERS), const),
            pl.BlockSpec((NUM_ADAPTERS, NUM_ADAPTERS * RANK), const),
            pl.BlockSpec((NUM_ADAPTERS * RANK, RANK), const),
            pl.BlockSpec((RANK, NUM_ADAPTERS * RANK), const),
        ],
        out_specs=pl.BlockSpec((TOKEN_TILE, D_OUT), lambda i: (i, 0)),
        out_shape=jax.ShapeDtypeStruct((b * s, D_OUT), jnp.float32),
    )(x2d, atf, wt, bft, bias2d, S, St, T, T2)
    return out.reshape(b, s, D_OUT)


# speedup vs baseline: 4.8270x; 1.0083x over previous
import functools

import jax
import jax.numpy as jnp
import numpy as np
from jax.experimental import pallas as pl

NUM_ADAPTERS = 16
RANK = 16
D_IN = 1024
D_OUT = 1024
TOPK = 4
TOKENS = 2048
TOKEN_TILE = 256


def _routewrap_kernel(x_ref, cat_ref, bft_ref, bias_ref, s_ref, st_ref,
                      t_ref, t2_ref, out_ref):
    xt = x_ref[:]
    P = jnp.dot(xt, cat_ref[:], preferred_element_type=jnp.float32)
    base = P[:, :D_OUT]
    V = P[:, D_OUT:]
    n2 = jnp.dot(V * V, s_ref[:], preferred_element_type=jnp.float32)

    iota = jax.lax.broadcasted_iota(jnp.int32, n2.shape, 1)
    cur = n2
    mask = jnp.zeros_like(n2)
    for _ in range(TOPK):
        m = jnp.max(cur, axis=-1, keepdims=True)
        eq = cur == m
        ci = jnp.min(jnp.where(eq, iota, NUM_ADAPTERS), axis=-1, keepdims=True)
        oh = iota == ci
        mask = jnp.where(oh, mask + 1.0, mask)
        cur = jnp.where(oh, -1.0, cur)
    routes = mask * (1.0 / TOPK)

    M = jnp.dot(routes, st_ref[:], preferred_element_type=jnp.float32)
    vs = jnp.dot(M * V, t_ref[:], preferred_element_type=jnp.float32)
    U = M * jnp.dot(vs, t2_ref[:], preferred_element_type=jnp.float32)

    acc = base + jnp.dot(U, bft_ref[:], preferred_element_type=jnp.float32)
    out_ref[:] = acc + bias_ref[:]


@jax.jit
def kernel(x, A, B, W, bias):
    b, s, _ = x.shape
    x2d = x.reshape(b * s, D_IN)
    atf = A.reshape(NUM_ADAPTERS * RANK, D_IN).T
    cat = jnp.concatenate([W.T, atf], axis=1)
    bft = B.transpose(0, 2, 1).reshape(NUM_ADAPTERS * RANK, D_OUT)
    bias2d = bias.reshape(1, D_OUT)

    blk = np.zeros((NUM_ADAPTERS * RANK, NUM_ADAPTERS), dtype=np.float32)
    blk[np.arange(NUM_ADAPTERS * RANK), np.arange(NUM_ADAPTERS * RANK) // RANK] = 1.0
    rnk = np.zeros((NUM_ADAPTERS * RANK, RANK), dtype=np.float32)
    rnk[np.arange(NUM_ADAPTERS * RANK), np.arange(NUM_ADAPTERS * RANK) % RANK] = 1.0
    S = jnp.asarray(blk)
    St = jnp.asarray(blk.T)
    T = jnp.asarray(rnk)
    T2 = jnp.asarray(rnk.T)

    n_tiles = (b * s) // TOKEN_TILE
    const = lambda i: (0, 0)
    out = pl.pallas_call(
        _routewrap_kernel,
        grid=(n_tiles,),
        in_specs=[
            pl.BlockSpec((TOKEN_TILE, D_IN), lambda i: (i, 0)),
            pl.BlockSpec((D_IN, D_OUT + NUM_ADAPTERS * RANK), const),
            pl.BlockSpec((NUM_ADAPTERS * RANK, D_OUT), const),
            pl.BlockSpec((1, D_OUT), const),
            pl.BlockSpec((NUM_ADAPTERS * RANK, NUM_ADAPTERS), const),
            pl.BlockSpec((NUM_ADAPTERS, NUM_ADAPTERS * RANK), const),
            pl.BlockSpec((NUM_ADAPTERS * RANK, RANK), const),
            pl.BlockSpec((RANK, NUM_ADAPTERS * RANK), const),
        ],
        out_specs=pl.BlockSpec((TOKEN_TILE, D_OUT), lambda i: (i, 0)),
        out_shape=jax.ShapeDtypeStruct((b * s, D_OUT), jnp.float32),
    )(x2d, cat, bft, bias2d, S, St, T, T2)
    return out.reshape(b, s, D_OUT)


# speedup vs baseline: 5.1280x; 1.0624x over previous
import functools

import jax
import jax.numpy as jnp
import numpy as np
from jax.experimental import pallas as pl

NUM_ADAPTERS = 16
RANK = 16
D_IN = 1024
D_OUT = 1024
TOPK = 4
TOKENS = 2048
TOKEN_TILE = 256


def _routewrap_kernel(x_ref, cat_ref, bft_ref, bias_ref, s_ref, st_ref,
                      t_ref, t2_ref, out_ref):
    xt = x_ref[:]
    P = jnp.dot(xt, cat_ref[:], preferred_element_type=jnp.float32)
    base = P[:, :D_OUT]
    V = P[:, D_OUT:]
    n2 = jnp.dot(V * V, s_ref[:], preferred_element_type=jnp.float32)

    iota = jax.lax.broadcasted_iota(jnp.int32, n2.shape, 1)
    key = (jax.lax.bitcast_convert_type(n2, jnp.int32) & ~15) | (15 - iota)
    cur = key
    for _ in range(TOPK - 1):
        m = jnp.max(cur, axis=-1, keepdims=True)
        cur = jnp.where(cur == m, -1, cur)
    m4 = jnp.max(cur, axis=-1, keepdims=True)
    routes = jnp.where(key >= m4, 1.0 / TOPK, 0.0)

    M = jnp.dot(routes, st_ref[:], preferred_element_type=jnp.float32)
    vs = jnp.dot(M * V, t_ref[:], preferred_element_type=jnp.float32)
    U = M * jnp.dot(vs, t2_ref[:], preferred_element_type=jnp.float32)

    acc = base + jnp.dot(U, bft_ref[:], preferred_element_type=jnp.float32)
    out_ref[:] = acc + bias_ref[:]


@jax.jit
def kernel(x, A, B, W, bias):
    b, s, _ = x.shape
    x2d = x.reshape(b * s, D_IN)
    atf = A.reshape(NUM_ADAPTERS * RANK, D_IN).T
    cat = jnp.concatenate([W.T, atf], axis=1)
    bft = B.transpose(0, 2, 1).reshape(NUM_ADAPTERS * RANK, D_OUT)
    bias2d = bias.reshape(1, D_OUT)

    blk = np.zeros((NUM_ADAPTERS * RANK, NUM_ADAPTERS), dtype=np.float32)
    blk[np.arange(NUM_ADAPTERS * RANK), np.arange(NUM_ADAPTERS * RANK) // RANK] = 1.0
    rnk = np.zeros((NUM_ADAPTERS * RANK, RANK), dtype=np.float32)
    rnk[np.arange(NUM_ADAPTERS * RANK), np.arange(NUM_ADAPTERS * RANK) % RANK] = 1.0
    S = jnp.asarray(blk)
    St = jnp.asarray(blk.T)
    T = jnp.asarray(rnk)
    T2 = jnp.asarray(rnk.T)

    n_tiles = (b * s) // TOKEN_TILE
    const = lambda i: (0, 0)
    out = pl.pallas_call(
        _routewrap_kernel,
        grid=(n_tiles,),
        in_specs=[
            pl.BlockSpec((TOKEN_TILE, D_IN), lambda i: (i, 0)),
            pl.BlockSpec((D_IN, D_OUT + NUM_ADAPTERS * RANK), const),
            pl.BlockSpec((NUM_ADAPTERS * RANK, D_OUT), const),
            pl.BlockSpec((1, D_OUT), const),
            pl.BlockSpec((NUM_ADAPTERS * RANK, NUM_ADAPTERS), const),
            pl.BlockSpec((NUM_ADAPTERS, NUM_ADAPTERS * RANK), const),
            pl.BlockSpec((NUM_ADAPTERS * RANK, RANK), const),
            pl.BlockSpec((RANK, NUM_ADAPTERS * RANK), const),
        ],
        out_specs=pl.BlockSpec((TOKEN_TILE, D_OUT), lambda i: (i, 0)),
        out_shape=jax.ShapeDtypeStruct((b * s, D_OUT), jnp.float32),
    )(x2d, cat, bft, bias2d, S, St, T, T2)
    return out.reshape(b, s, D_OUT)
